# speedup vs baseline: 2.3677x; 2.3677x over previous
import functools
import math

import jax
import jax.numpy as jnp
from jax import lax
from jax.experimental import pallas as pl
from jax.experimental.pallas import tpu as pltpu
from jax.experimental.pallas import tpu_sc as plsc

D_MODEL = 128
SCALE = math.sqrt(float(D_MODEL))
LANES = 16

NUM_CORES = 2
NUM_SUBCORES = 16
NUM_WORKERS = NUM_CORES * NUM_SUBCORES

B_TOTAL = 4096 * 50
B_PER_W = B_TOTAL // NUM_WORKERS
CHUNK = 160
NCHUNKS = B_PER_W // CHUNK


_mesh = plsc.VectorSubcoreMesh(core_axis_name="c", subcore_axis_name="s")


@functools.partial(
    pl.kernel,
    out_type=jax.ShapeDtypeStruct((B_TOTAL, D_MODEL), jnp.float32),
    mesh=_mesh,
    scratch_types=[
        pltpu.VMEM((CHUNK,), jnp.int32),
        pltpu.VMEM((CHUNK, D_MODEL), jnp.float32),
        pltpu.SemaphoreType.DMA,
    ],
)
def _embed(idx_hbm, table_hbm, out_hbm, idx_v, rows_v, sem):
    wid = lax.axis_index("s") * NUM_CORES + lax.axis_index("c")
    base = wid * B_PER_W

    def chunk_body(g, carry):
        off = base + g * CHUNK
        pltpu.sync_copy(idx_hbm.at[pl.ds(off, CHUNK)], idx_v)
        pltpu.async_copy(table_hbm.at[idx_v], rows_v, sem).wait()

        def scale_row(r, c):
            for j in range(D_MODEL // LANES):
                sl = pl.ds(j * LANES, LANES)
                rows_v[r, sl] = rows_v[r, sl] * SCALE
            return c

        lax.fori_loop(0, CHUNK, scale_row, 0)
        pltpu.sync_copy(rows_v, out_hbm.at[pl.ds(off, CHUNK)])
        return carry

    lax.fori_loop(0, NCHUNKS, chunk_body, 0)


def kernel(x, table):
    idx = x.reshape(-1).astype(jnp.int32)
    out = _embed(idx, table)
    return out.reshape(x.shape + (D_MODEL,))


# speedup vs baseline: 2.9092x; 1.2287x over previous
import functools
import math

import jax
import jax.numpy as jnp
from jax import lax
from jax.experimental import pallas as pl
from jax.experimental.pallas import tpu as pltpu
from jax.experimental.pallas import tpu_sc as plsc

D_MODEL = 128
SCALE = math.sqrt(float(D_MODEL))
LANES = 16

NUM_CORES = 2
NUM_SUBCORES = 16
NUM_WORKERS = NUM_CORES * NUM_SUBCORES

B_TOTAL = 4096 * 50
B_PER_W = B_TOTAL // NUM_WORKERS
NBUF = 2
CHUNK = 200
NCHUNKS = B_PER_W // CHUNK
NGROUP = NCHUNKS // NBUF


_mesh = plsc.VectorSubcoreMesh(core_axis_name="c", subcore_axis_name="s")


@functools.partial(
    pl.kernel,
    out_type=jax.ShapeDtypeStruct((B_TOTAL, D_MODEL), jnp.float32),
    mesh=_mesh,
    scratch_types=[
        pltpu.VMEM((B_PER_W,), jnp.int32),
        pltpu.VMEM((NBUF, CHUNK, D_MODEL), jnp.float32),
        pltpu.VMEM((NBUF, CHUNK, D_MODEL), jnp.float32),
        pltpu.SemaphoreType.DMA((NBUF,)),
        pltpu.SemaphoreType.DMA((NBUF,)),
    ],
)
def _embed(idx_hbm, table_hbm, out_hbm, idx_all, rin, rout, sem_g, sem_o):
    wid = lax.axis_index("s") * NUM_CORES + lax.axis_index("c")
    base = wid * B_PER_W
    pltpu.sync_copy(idx_hbm.at[pl.ds(base, B_PER_W)], idx_all)

    def gather_start(g, b):
        pltpu.async_copy(
            table_hbm.at[idx_all.at[pl.ds(g * CHUNK, CHUNK)]],
            rin.at[b], sem_g.at[b])

    def gather_wait(b):
        pltpu.make_async_copy(
            table_hbm.at[idx_all.at[pl.ds(0, CHUNK)]],
            rin.at[b], sem_g.at[b]).wait()

    def out_start(g, b):
        pltpu.async_copy(
            rout.at[b], out_hbm.at[pl.ds(base + g * CHUNK, CHUNK)],
            sem_o.at[b])

    def out_wait(b):
        pltpu.make_async_copy(
            rout.at[b], out_hbm.at[pl.ds(base, CHUNK)], sem_o.at[b]).wait()

    gather_start(0, 0)

    def group_body(p, carry):
        for b in range(NBUF):
            g = p * NBUF + b
            gather_wait(b)

            @pl.when(g + 1 < NCHUNKS)
            def _():
                gather_start(g + 1, (b + 1) % NBUF)

            @pl.when(g >= NBUF)
            def _():
                out_wait(b)

            @plsc.parallel_loop(0, CHUNK, unroll=4)
            def _(r):
                for j in range(D_MODEL // LANES):
                    sl = pl.ds(j * LANES, LANES)
                    rout[b, r, sl] = rin[b, r, sl] * SCALE

            out_start(g, b)
        return carry

    lax.fori_loop(0, NGROUP, group_body, 0)
    for b in range(NBUF):
        out_wait(b)


def kernel(x, table):
    idx = x.reshape(-1).astype(jnp.int32)
    out = _embed(idx, table)
    return out.reshape(x.shape + (D_MODEL,))


# speedup vs baseline: 2.9377x; 1.0098x over previous
import functools
import math

import jax
import jax.numpy as jnp
from jax import lax
from jax.experimental import pallas as pl
from jax.experimental.pallas import tpu as pltpu
from jax.experimental.pallas import tpu_sc as plsc

D_MODEL = 128
SCALE = math.sqrt(float(D_MODEL))
LANES = 16

NUM_CORES = 2
NUM_SUBCORES = 16
NUM_WORKERS = NUM_CORES * NUM_SUBCORES

B_TOTAL = 4096 * 50
B_PER_W = B_TOTAL // NUM_WORKERS
NBUF = 4
CHUNK = 160
NCHUNKS = B_PER_W // CHUNK
NGROUP = NCHUNKS // NBUF
LOOKAHEAD = NBUF - 1


_mesh = plsc.VectorSubcoreMesh(core_axis_name="c", subcore_axis_name="s")


@functools.partial(
    pl.kernel,
    out_type=jax.ShapeDtypeStruct((B_TOTAL, D_MODEL), jnp.float32),
    mesh=_mesh,
    scratch_types=[
        pltpu.VMEM((B_PER_W,), jnp.int32),
        pltpu.VMEM((NBUF, CHUNK, D_MODEL), jnp.float32),
        pltpu.SemaphoreType.DMA((NBUF,)),
        pltpu.SemaphoreType.DMA((NBUF,)),
    ],
)
def _embed(idx_hbm, table_hbm, out_hbm, idx_all, rows, sem_g, sem_o):
    wid = lax.axis_index("s") * NUM_CORES + lax.axis_index("c")
    base = wid * B_PER_W
    pltpu.sync_copy(idx_hbm.at[pl.ds(base, B_PER_W)], idx_all)

    def gather_start(g, b):
        pltpu.async_copy(
            table_hbm.at[idx_all.at[pl.ds(g * CHUNK, CHUNK)]],
            rows.at[b], sem_g.at[b])

    def gather_wait(b):
        pltpu.make_async_copy(
            table_hbm.at[idx_all.at[pl.ds(0, CHUNK)]],
            rows.at[b], sem_g.at[b]).wait()

    def out_start(g, b):
        pltpu.async_copy(
            rows.at[b], out_hbm.at[pl.ds(base + g * CHUNK, CHUNK)],
            sem_o.at[b])

    def out_wait(b):
        pltpu.make_async_copy(
            rows.at[b], out_hbm.at[pl.ds(base, CHUNK)], sem_o.at[b]).wait()

    for g in range(LOOKAHEAD):
        gather_start(g, g)

    def group_body(p, carry):
        for b in range(NBUF):
            g = p * NBUF + b
            gather_wait(b)

            @plsc.parallel_loop(0, CHUNK, unroll=8)
            def _(r):
                for j in range(D_MODEL // LANES):
                    sl = pl.ds(j * LANES, LANES)
                    rows[b, r, sl] = rows[b, r, sl] * SCALE

            out_start(g, b)

            bn = (b + LOOKAHEAD) % NBUF
            gp = g + LOOKAHEAD

            @pl.when((gp < NCHUNKS) & (g >= 1))
            def _():
                out_wait(bn)

            @pl.when(gp < NCHUNKS)
            def _():
                gather_start(gp, bn)
        return carry

    lax.fori_loop(0, NGROUP, group_body, 0)
    for b in range(NBUF):
        out_wait(b)


def kernel(x, table):
    idx = x.reshape(-1).astype(jnp.int32)
    out = _embed(idx, table)
    return out.reshape(x.shape + (D_MODEL,))


# speedup vs baseline: 5.0943x; 1.7341x over previous
import functools
import math

import jax
import jax.numpy as jnp
from jax import lax
from jax.experimental import pallas as pl
from jax.experimental.pallas import tpu as pltpu
from jax.experimental.pallas import tpu_sc as plsc

D_MODEL = 128
SCALE = math.sqrt(float(D_MODEL))
LANES = 16

NUM_CORES = 2
NUM_SUBCORES = 16
NUM_WORKERS = NUM_CORES * NUM_SUBCORES

N_SEQ = 4096
TOK = 50
NI = N_SEQ // NUM_WORKERS
NBUF = 4
LOOKAHEAD = NBUF - 1
NGROUP = NI // NBUF


_mesh = plsc.VectorSubcoreMesh(core_axis_name="c", subcore_axis_name="s")


@functools.partial(
    pl.kernel,
    out_type=jax.ShapeDtypeStruct((N_SEQ, TOK, D_MODEL), jnp.float32),
    mesh=_mesh,
    compiler_params=pltpu.CompilerParams(use_tc_tiling_on_sc=True),
    scratch_types=[
        pltpu.VMEM((NI, TOK), jnp.int32),
        pltpu.VMEM((NBUF, TOK, D_MODEL), jnp.float32),
        pltpu.SemaphoreType.DMA((NBUF,)),
        pltpu.SemaphoreType.DMA((NBUF,)),
    ],
)
def _embed(x_hbm, table_hbm, out_hbm, idx_v, rows, sem_g, sem_o):
    wid = lax.axis_index("s") * NUM_CORES + lax.axis_index("c")
    i0 = wid * NI
    pltpu.sync_copy(x_hbm.at[pl.ds(i0, NI), :], idx_v)

    def gather_start(s, b):
        pltpu.async_copy(
            table_hbm.at[idx_v.at[s]], rows.at[b], sem_g.at[b])

    def gather_wait(b):
        pltpu.make_async_copy(
            table_hbm.at[idx_v.at[0]], rows.at[b], sem_g.at[b]).wait()

    def out_start(s, b):
        pltpu.async_copy(rows.at[b], out_hbm.at[i0 + s], sem_o.at[b])

    def out_wait(b):
        pltpu.make_async_copy(
            rows.at[b], out_hbm.at[i0], sem_o.at[b]).wait()

    for s in range(LOOKAHEAD):
        gather_start(s, s)

    def group_body(p, carry):
        for b in range(NBUF):
            s = p * NBUF + b
            gather_wait(b)

            @plsc.parallel_loop(0, TOK, unroll=2)
            def _(r):
                for j in range(D_MODEL // LANES):
                    sl = pl.ds(j * LANES, LANES)
                    rows[b, r, sl] = rows[b, r, sl] * SCALE

            out_start(s, b)

            bn = (b + LOOKAHEAD) % NBUF
            sp = s + LOOKAHEAD

            @pl.when((sp < NI) & (s >= 1))
            def _():
                out_wait(bn)

            @pl.when(sp < NI)
            def _():
                gather_start(sp, bn)
        return carry

    lax.fori_loop(0, NGROUP, group_body, 0)
    for b in range(NBUF):
        out_wait(b)


def kernel(x, table):
    return _embed(x.astype(jnp.int32), table)


# speedup vs baseline: 9.3068x; 1.8269x over previous
import functools
import math

import jax
import jax.numpy as jnp
from jax import lax
from jax.experimental import pallas as pl
from jax.experimental.pallas import tpu as pltpu
from jax.experimental.pallas import tpu_sc as plsc

D_MODEL = 128
SCALE = math.sqrt(float(D_MODEL))
LANES = 16

NUM_CORES = 2
NUM_SUBCORES = 16
NUM_WORKERS = NUM_CORES * NUM_SUBCORES

N_SEQ = 4096
TOK = 50
NI = N_SEQ // NUM_WORKERS
NBUF = 5
LOOKAHEAD = NBUF - 1
NGROUP = TOK // NBUF


_mesh = plsc.VectorSubcoreMesh(core_axis_name="c", subcore_axis_name="s")


@functools.partial(
    pl.kernel,
    out_type=jax.ShapeDtypeStruct((TOK, N_SEQ, D_MODEL), jnp.float32),
    mesh=_mesh,
    compiler_params=pltpu.CompilerParams(use_tc_tiling_on_sc=True),
    scratch_types=[
        pltpu.VMEM((TOK, NI), jnp.int32),
        pltpu.VMEM((NBUF, NI, D_MODEL), jnp.float32),
        pltpu.SemaphoreType.DMA((NBUF,)),
        pltpu.SemaphoreType.DMA((NBUF,)),
    ],
)
def _embed(xt_hbm, table_hbm, out_hbm, idx_v, rows, sem_g, sem_o):
    wid = lax.axis_index("s") * NUM_CORES + lax.axis_index("c")
    i0 = wid * NI
    pltpu.sync_copy(xt_hbm.at[:, pl.ds(i0, NI)], idx_v)

    def gather_start(j, b):
        pltpu.async_copy(
            table_hbm.at[idx_v.at[j]], rows.at[b], sem_g.at[b])

    def gather_wait(b):
        pltpu.make_async_copy(
            table_hbm.at[idx_v.at[0]], rows.at[b], sem_g.at[b]).wait()

    def out_start(j, b):
        pltpu.async_copy(
            rows.at[b], out_hbm.at[j, pl.ds(i0, NI), :], sem_o.at[b])

    def out_wait(b):
        pltpu.make_async_copy(
            rows.at[b], out_hbm.at[0, pl.ds(i0, NI), :], sem_o.at[b]).wait()

    for j in range(LOOKAHEAD):
        gather_start(j, j)

    def group_body(p, carry):
        for b in range(NBUF):
            j = p * NBUF + b
            gather_wait(b)

            @plsc.parallel_loop(0, NI, unroll=4)
            def _(r):
                for q in range(D_MODEL // LANES):
                    sl = pl.ds(q * LANES, LANES)
                    rows[b, r, sl] = rows[b, r, sl] * SCALE

            out_start(j, b)

            bn = (b + LOOKAHEAD) % NBUF
            jp = j + LOOKAHEAD

            @pl.when((jp < TOK) & (j >= 1))
            def _():
                out_wait(bn)

            @pl.when(jp < TOK)
            def _():
                gather_start(jp, bn)
        return carry

    lax.fori_loop(0, NGROUP, group_body, 0)
    for b in range(NBUF):
        out_wait(b)


def kernel(x, table):
    xt = jnp.transpose(x.astype(jnp.int32))
    out = _embed(xt, table)
    return jnp.transpose(out, (1, 0, 2))
